# baseline (speedup 1.0000x reference)
import jax
import jax.numpy as jnp
from jax.experimental import pallas as pl
from jax.experimental.pallas import tpu as pltpu

D = 256
N_STUDENTS = 10000
N_Q = 2048
BM = 1432


NSPLIT = 4
KS = N_Q // NSPLIT


def _gcn_body(hq_ref, wt_ref, bias_ref, a1_ref, a2_ref, a3_ref, a4_ref,
              emb_ref, gamma_ref, beta_ref, out_ref, wh_ref):
    i = pl.program_id(0)

    @pl.when(i == 0)
    def _():
        wh_ref[:] = (
            jnp.dot(hq_ref[:], wt_ref[:], preferred_element_type=jnp.float32)
            + bias_ref[:]
        )

    a_refs = (a1_ref, a2_ref, a3_ref, a4_ref)
    deg = jnp.zeros((BM, 1), jnp.float32)
    acc = jnp.zeros((BM, D), jnp.float32)
    for k, ar in enumerate(a_refs):
        a = ar[:]
        deg = deg + jnp.sum(a, axis=1, keepdims=True)
        acc = acc + jnp.dot(a, wh_ref[k * KS : (k + 1) * KS],
                            preferred_element_type=jnp.float32)
    inv_deg = 1.0 / jnp.maximum(deg, 1.0)
    x = emb_ref[:] + jnp.maximum(acc, 0.0) * inv_deg
    mean = jnp.mean(x, axis=1, keepdims=True)
    xc = x - mean
    var = jnp.mean(xc * xc, axis=1, keepdims=True)
    out_ref[:] = xc * jax.lax.rsqrt(var + 1e-5) * gamma_ref[:] + beta_ref[:]


def kernel(H_q, A_uq, emb_weight, W_weight, W_bias, ln_gamma, ln_beta):
    grid = ((N_STUDENTS + BM - 1) // BM,)
    const = lambda i: (0, 0)
    return pl.pallas_call(
        _gcn_body,
        grid=grid,
        in_specs=[
            pl.BlockSpec((N_Q, D), const),
            pl.BlockSpec((D, D), const),
            pl.BlockSpec((1, D), const),
            pl.BlockSpec((BM, KS), lambda i: (i, 0)),
            pl.BlockSpec((BM, KS), lambda i: (i, 1)),
            pl.BlockSpec((BM, KS), lambda i: (i, 2)),
            pl.BlockSpec((BM, KS), lambda i: (i, 3)),
            pl.BlockSpec((BM, D), lambda i: (i, 0)),
            pl.BlockSpec((1, D), const),
            pl.BlockSpec((1, D), const),
        ],
        out_specs=pl.BlockSpec((BM, D), lambda i: (i, 0)),
        out_shape=jax.ShapeDtypeStruct((N_STUDENTS, D), jnp.float32),
        scratch_shapes=[pltpu.VMEM((N_Q, D), jnp.float32)],
    )(
        H_q,
        W_weight.T,
        W_bias.reshape(1, D),
        A_uq,
        A_uq,
        A_uq,
        A_uq,
        emb_weight,
        ln_gamma.reshape(1, D),
        ln_beta.reshape(1, D),
    )


# speedup vs baseline: 1.0038x; 1.0038x over previous
import jax
import jax.numpy as jnp
from jax.experimental import pallas as pl
from jax.experimental.pallas import tpu as pltpu

D = 256
N_STUDENTS = 10000
N_Q = 2048
BM = 1112


NSPLIT = 4
KS = N_Q // NSPLIT


def _gcn_body(hq_ref, wt_ref, bias_ref, a1_ref, a2_ref, a3_ref, a4_ref,
              emb_ref, gamma_ref, beta_ref, out_ref, wh_ref):
    i = pl.program_id(0)

    @pl.when(i == 0)
    def _():
        wh_ref[:] = (
            jnp.dot(hq_ref[:], wt_ref[:], preferred_element_type=jnp.float32)
            + bias_ref[:]
        )

    a_refs = (a1_ref, a2_ref, a3_ref, a4_ref)
    deg = jnp.zeros((BM, 1), jnp.float32)
    acc = jnp.zeros((BM, D), jnp.float32)
    for k, ar in enumerate(a_refs):
        a = ar[:]
        deg = deg + jnp.sum(a, axis=1, keepdims=True)
        acc = acc + jnp.dot(a, wh_ref[k * KS : (k + 1) * KS],
                            preferred_element_type=jnp.float32)
    inv_deg = 1.0 / jnp.maximum(deg, 1.0)
    x = emb_ref[:] + jnp.maximum(acc, 0.0) * inv_deg
    mean = jnp.mean(x, axis=1, keepdims=True)
    xc = x - mean
    var = jnp.mean(xc * xc, axis=1, keepdims=True)
    out_ref[:] = xc * jax.lax.rsqrt(var + 1e-5) * gamma_ref[:] + beta_ref[:]


def kernel(H_q, A_uq, emb_weight, W_weight, W_bias, ln_gamma, ln_beta):
    grid = ((N_STUDENTS + BM - 1) // BM,)
    const = lambda i: (0, 0)
    return pl.pallas_call(
        _gcn_body,
        grid=grid,
        in_specs=[
            pl.BlockSpec((N_Q, D), const),
            pl.BlockSpec((D, D), const),
            pl.BlockSpec((1, D), const),
            pl.BlockSpec((BM, KS), lambda i: (i, 0)),
            pl.BlockSpec((BM, KS), lambda i: (i, 1)),
            pl.BlockSpec((BM, KS), lambda i: (i, 2)),
            pl.BlockSpec((BM, KS), lambda i: (i, 3)),
            pl.BlockSpec((BM, D), lambda i: (i, 0)),
            pl.BlockSpec((1, D), const),
            pl.BlockSpec((1, D), const),
        ],
        out_specs=pl.BlockSpec((BM, D), lambda i: (i, 0)),
        out_shape=jax.ShapeDtypeStruct((N_STUDENTS, D), jnp.float32),
        scratch_shapes=[pltpu.VMEM((N_Q, D), jnp.float32)],
    )(
        H_q,
        W_weight.T,
        W_bias.reshape(1, D),
        A_uq,
        A_uq,
        A_uq,
        A_uq,
        emb_weight,
        ln_gamma.reshape(1, D),
        ln_beta.reshape(1, D),
    )


# speedup vs baseline: 1.0103x; 1.0065x over previous
import jax
import jax.numpy as jnp
from jax.experimental import pallas as pl
from jax.experimental.pallas import tpu as pltpu

D = 256
N_STUDENTS = 10000
N_Q = 2048
BM = 1256


NSPLIT = 2
KS = N_Q // NSPLIT


def _gcn_body(hq_ref, wt_ref, bias_ref, a1_ref, a2_ref,
              emb_ref, gamma_ref, beta_ref, out_ref, wh_ref):
    i = pl.program_id(0)

    @pl.when(i == 0)
    def _():
        wh_ref[:] = (
            jnp.dot(hq_ref[:], wt_ref[:], preferred_element_type=jnp.float32)
            + bias_ref[:]
        )

    a_refs = (a1_ref, a2_ref)
    deg = jnp.zeros((BM, 1), jnp.float32)
    acc = jnp.zeros((BM, D), jnp.float32)
    for k, ar in enumerate(a_refs):
        a = ar[:]
        deg = deg + jnp.sum(a, axis=1, keepdims=True)
        acc = acc + jnp.dot(a, wh_ref[k * KS : (k + 1) * KS],
                            preferred_element_type=jnp.float32)
    inv_deg = 1.0 / jnp.maximum(deg, 1.0)
    x = emb_ref[:] + jnp.maximum(acc, 0.0) * inv_deg
    mean = jnp.mean(x, axis=1, keepdims=True)
    xc = x - mean
    var = jnp.mean(xc * xc, axis=1, keepdims=True)
    out_ref[:] = xc * jax.lax.rsqrt(var + 1e-5) * gamma_ref[:] + beta_ref[:]


def kernel(H_q, A_uq, emb_weight, W_weight, W_bias, ln_gamma, ln_beta):
    grid = ((N_STUDENTS + BM - 1) // BM,)
    const = lambda i: (0, 0)
    return pl.pallas_call(
        _gcn_body,
        grid=grid,
        in_specs=[
            pl.BlockSpec((N_Q, D), const),
            pl.BlockSpec((D, D), const),
            pl.BlockSpec((1, D), const),
            pl.BlockSpec((BM, KS), lambda i: (i, 0)),
            pl.BlockSpec((BM, KS), lambda i: (i, 1)),
            pl.BlockSpec((BM, D), lambda i: (i, 0)),
            pl.BlockSpec((1, D), const),
            pl.BlockSpec((1, D), const),
        ],
        out_specs=pl.BlockSpec((BM, D), lambda i: (i, 0)),
        out_shape=jax.ShapeDtypeStruct((N_STUDENTS, D), jnp.float32),
        scratch_shapes=[pltpu.VMEM((N_Q, D), jnp.float32)],
    )(
        H_q,
        W_weight.T,
        W_bias.reshape(1, D),
        A_uq,
        A_uq,
        emb_weight,
        ln_gamma.reshape(1, D),
        ln_beta.reshape(1, D),
    )
